# baseline (speedup 1.0000x reference)
import functools

import jax
import jax.numpy as jnp
from jax import lax
from jax.experimental import pallas as pl
from jax.experimental.pallas import tpu as pltpu
from jax.experimental.pallas import tpu_sc as plsc

B, C_IN, C_OUT, N, K = 1, 128, 128, 10000, 32
DEG = K + 1

NSUB = 16
NPT_SLOW = 160
NPT_FAST = 480
SLOW_CID = 0
N_PAD = NSUB * (NPT_SLOW + NPT_FAST)
GROUP = 2
NBUF = 4
LANES = 16
CVECS = C_OUT // LANES
GMAX = NPT_FAST // GROUP
TC_BLK = 1024


def _tc_body(x_ref, w_ref, b_ref, o_ref):
    acc = lax.dot_general(
        x_ref[:, :], w_ref[:, :],
        (((0,), (1,)), ((), ())),
        preferred_element_type=jnp.float32,
    )
    o_ref[:, :] = jnp.maximum(acc, 0.0) + b_ref[:, :]


def _compute_h(x_pad, w_scaled, b_scaled):
    return pl.pallas_call(
        _tc_body,
        grid=(N_PAD // TC_BLK,),
        in_specs=[
            pl.BlockSpec((C_IN, TC_BLK), lambda i: (0, i)),
            pl.BlockSpec((C_OUT, C_IN), lambda i: (0, 0)),
            pl.BlockSpec((1, C_OUT), lambda i: (0, 0)),
        ],
        out_specs=pl.BlockSpec((TC_BLK, C_OUT), lambda i: (i, 0)),
        out_shape=jax.ShapeDtypeStruct((N_PAD, C_OUT), jnp.float32),
    )(x_pad, w_scaled, b_scaled)


def _worker(h_hbm, ei_hbm, out_hbm, idx_v, acc_v, rows, gsems,
            node_base, ngroups, is_slow):
    node_base = pl.multiple_of(node_base, 8)
    rowbase = pl.multiple_of(node_base // GROUP, 8)

    pltpu.sync_copy(ei_hbm.at[pl.ds(rowbase, GMAX)], idx_v)
    pltpu.sync_copy(h_hbm.at[pl.ds(node_base, NPT_FAST)], acc_v)

    def gather(g, b):
        pltpu.async_copy(h_hbm.at[idx_v.at[g]], rows[b], gsems[b])

    def wait_gather(g, b):
        pltpu.make_async_copy(h_hbm.at[idx_v.at[g]], rows[b],
                              gsems[b]).wait()

    def consume(g, b):
        rows_b = rows[b]

        def node(n_, carry):
            row = g * GROUP + n_
            base = n_ * K
            for c in range(CVECS):
                cs = pl.ds(c * LANES, LANES)
                s = acc_v[row, cs]
                for r in range(K):
                    s = s + rows_b[base + r, cs]
                acc_v[row, cs] = s
            return carry

        lax.fori_loop(0, GROUP, node, 0)

    for b in range(NBUF):
        gather(b, b)

    def body(o, carry):
        g0 = o * NBUF
        for b in range(NBUF):
            g = g0 + b
            wait_gather(g, b)
            consume(g, b)
            gather(g + NBUF, b)
        return carry

    lax.fori_loop(0, ngroups // NBUF - 1, body, 0)

    g0 = ngroups - NBUF
    for b in range(NBUF):
        wait_gather(g0 + b, b)
        consume(g0 + b, b)

    @pl.when(is_slow)
    def _():
        pltpu.sync_copy(acc_v.at[pl.ds(0, NPT_SLOW)],
                        out_hbm.at[pl.ds(node_base, NPT_SLOW)])

    @pl.when(jnp.logical_not(is_slow))
    def _():
        pltpu.sync_copy(acc_v, out_hbm.at[pl.ds(node_base, NPT_FAST)])


def _sc_body(h_hbm, ei_hbm, out_hbm,
             idx_v, acc_v, rows0_v, rows1_v, rows2_v, rows3_v,
             gsem0, gsem1, gsem2, gsem3):
    cid = lax.axis_index("c")
    sid = lax.axis_index("s")
    rows = (rows0_v, rows1_v, rows2_v, rows3_v)
    gsems = (gsem0, gsem1, gsem2, gsem3)

    slow_first = SLOW_CID == 0
    base_slow = sid * NPT_SLOW if slow_first else \
        NSUB * NPT_FAST + sid * NPT_SLOW
    base_fast = NSUB * NPT_SLOW + sid * NPT_FAST if slow_first else \
        sid * NPT_FAST

    is_slow = cid == SLOW_CID
    node_base = jnp.where(is_slow, base_slow, base_fast)
    ngroups = jnp.where(is_slow, NPT_SLOW // GROUP, NPT_FAST // GROUP)
    _worker(h_hbm, ei_hbm, out_hbm, idx_v, acc_v, rows, gsems,
            node_base, ngroups, is_slow)


@functools.partial(
    pl.kernel,
    out_type=jax.ShapeDtypeStruct((N_PAD, C_OUT), jnp.float32),
    mesh=plsc.VectorSubcoreMesh(core_axis_name="c", subcore_axis_name="s"),
    scratch_types=[
        pltpu.VMEM((GMAX, GROUP * K), jnp.int32),
        pltpu.VMEM((NPT_FAST, C_OUT), jnp.float32),
        pltpu.VMEM((GROUP * K, C_OUT), jnp.float32),
        pltpu.VMEM((GROUP * K, C_OUT), jnp.float32),
        pltpu.VMEM((GROUP * K, C_OUT), jnp.float32),
        pltpu.VMEM((GROUP * K, C_OUT), jnp.float32),
        pltpu.SemaphoreType.DMA,
        pltpu.SemaphoreType.DMA,
        pltpu.SemaphoreType.DMA,
        pltpu.SemaphoreType.DMA,
    ],
)
def _sc_aggregate(h_hbm, ei_hbm, out_hbm, *scratch):
    _sc_body(h_hbm, ei_hbm, out_hbm, *scratch)


def kernel(x, edge_index, W, bias):
    x2 = x[0, :, :, 0]
    x_pad = jnp.pad(x2, ((0, 0), (0, N_PAD - N)))
    w_scaled = W * jnp.float32(1.0 / DEG)
    b_scaled = (bias[0, :, 0, 0] * jnp.float32(1.0 / DEG)).reshape(1, C_OUT)

    h = _compute_h(x_pad, w_scaled, b_scaled)

    ei = edge_index[0, 0].astype(jnp.int32)
    ei_pad = jnp.pad(ei, ((0, N_PAD - N), (0, 0)))
    ei_groups = ei_pad.reshape(N_PAD // GROUP, GROUP * K)

    out_pad = _sc_aggregate(h, ei_groups)

    out = out_pad[:N].T
    return out.reshape(1, C_OUT, N, 1)


# speedup vs baseline: 1.1867x; 1.1867x over previous
import functools

import jax
import jax.numpy as jnp
from jax import lax
from jax.experimental import pallas as pl
from jax.experimental.pallas import tpu as pltpu
from jax.experimental.pallas import tpu_sc as plsc

B, C_IN, C_OUT, N, K = 1, 128, 128, 10000, 32
DEG = K + 1

NSUB = 16
NPT_SLOW = 160
NPT_FAST = 480
SLOW_CID = 1
N_PAD = NSUB * (NPT_SLOW + NPT_FAST)
GROUP = 2
NBUF = 4
LANES = 16
CVECS = C_OUT // LANES
GMAX = NPT_FAST // GROUP
TC_BLK = 1024


def _tc_body(x_ref, w_ref, b_ref, o_ref):
    acc = lax.dot_general(
        x_ref[:, :], w_ref[:, :],
        (((0,), (1,)), ((), ())),
        preferred_element_type=jnp.float32,
    )
    o_ref[:, :] = jnp.maximum(acc, 0.0) + b_ref[:, :]


def _compute_h(x_pad, w_scaled, b_scaled):
    return pl.pallas_call(
        _tc_body,
        grid=(N_PAD // TC_BLK,),
        in_specs=[
            pl.BlockSpec((C_IN, TC_BLK), lambda i: (0, i)),
            pl.BlockSpec((C_OUT, C_IN), lambda i: (0, 0)),
            pl.BlockSpec((1, C_OUT), lambda i: (0, 0)),
        ],
        out_specs=pl.BlockSpec((TC_BLK, C_OUT), lambda i: (i, 0)),
        out_shape=jax.ShapeDtypeStruct((N_PAD, C_OUT), jnp.float32),
    )(x_pad, w_scaled, b_scaled)


def _worker(h_hbm, ei_hbm, out_hbm, idx_v, acc_v, rows, gsems,
            node_base, ngroups, is_slow):
    node_base = pl.multiple_of(node_base, 8)
    rowbase = pl.multiple_of(node_base // GROUP, 8)

    pltpu.sync_copy(ei_hbm.at[pl.ds(rowbase, GMAX)], idx_v)
    pltpu.sync_copy(h_hbm.at[pl.ds(node_base, NPT_FAST)], acc_v)

    def gather(g, b):
        pltpu.async_copy(h_hbm.at[idx_v.at[g]], rows[b], gsems[b])

    def wait_gather(g, b):
        pltpu.make_async_copy(h_hbm.at[idx_v.at[g]], rows[b],
                              gsems[b]).wait()

    def consume(g, b):
        rows_b = rows[b]

        def node(n_, carry):
            row = g * GROUP + n_
            base = n_ * K
            for c in range(CVECS):
                cs = pl.ds(c * LANES, LANES)
                s = acc_v[row, cs]
                for r in range(K):
                    s = s + rows_b[base + r, cs]
                acc_v[row, cs] = s
            return carry

        lax.fori_loop(0, GROUP, node, 0)

    for b in range(NBUF):
        gather(b, b)

    def body(o, carry):
        g0 = o * NBUF
        for b in range(NBUF):
            g = g0 + b
            wait_gather(g, b)
            consume(g, b)
            gather(g + NBUF, b)
        return carry

    lax.fori_loop(0, ngroups // NBUF - 1, body, 0)

    g0 = ngroups - NBUF
    for b in range(NBUF):
        wait_gather(g0 + b, b)
        consume(g0 + b, b)

    @pl.when(is_slow)
    def _():
        pltpu.sync_copy(acc_v.at[pl.ds(0, NPT_SLOW)],
                        out_hbm.at[pl.ds(node_base, NPT_SLOW)])

    @pl.when(jnp.logical_not(is_slow))
    def _():
        pltpu.sync_copy(acc_v, out_hbm.at[pl.ds(node_base, NPT_FAST)])


def _sc_body(h_hbm, ei_hbm, out_hbm,
             idx_v, acc_v, rows0_v, rows1_v, rows2_v, rows3_v,
             gsem0, gsem1, gsem2, gsem3):
    cid = lax.axis_index("c")
    sid = lax.axis_index("s")
    rows = (rows0_v, rows1_v, rows2_v, rows3_v)
    gsems = (gsem0, gsem1, gsem2, gsem3)

    slow_first = SLOW_CID == 0
    base_slow = sid * NPT_SLOW if slow_first else \
        NSUB * NPT_FAST + sid * NPT_SLOW
    base_fast = NSUB * NPT_SLOW + sid * NPT_FAST if slow_first else \
        sid * NPT_FAST

    is_slow = cid == SLOW_CID
    node_base = jnp.where(is_slow, base_slow, base_fast)
    ngroups = jnp.where(is_slow, NPT_SLOW // GROUP, NPT_FAST // GROUP)
    _worker(h_hbm, ei_hbm, out_hbm, idx_v, acc_v, rows, gsems,
            node_base, ngroups, is_slow)


@functools.partial(
    pl.kernel,
    out_type=jax.ShapeDtypeStruct((N_PAD, C_OUT), jnp.float32),
    mesh=plsc.VectorSubcoreMesh(core_axis_name="c", subcore_axis_name="s"),
    scratch_types=[
        pltpu.VMEM((GMAX, GROUP * K), jnp.int32),
        pltpu.VMEM((NPT_FAST, C_OUT), jnp.float32),
        pltpu.VMEM((GROUP * K, C_OUT), jnp.float32),
        pltpu.VMEM((GROUP * K, C_OUT), jnp.float32),
        pltpu.VMEM((GROUP * K, C_OUT), jnp.float32),
        pltpu.VMEM((GROUP * K, C_OUT), jnp.float32),
        pltpu.SemaphoreType.DMA,
        pltpu.SemaphoreType.DMA,
        pltpu.SemaphoreType.DMA,
        pltpu.SemaphoreType.DMA,
    ],
)
def _sc_aggregate(h_hbm, ei_hbm, out_hbm, *scratch):
    _sc_body(h_hbm, ei_hbm, out_hbm, *scratch)


def kernel(x, edge_index, W, bias):
    x2 = x[0, :, :, 0]
    x_pad = jnp.pad(x2, ((0, 0), (0, N_PAD - N)))
    w_scaled = W * jnp.float32(1.0 / DEG)
    b_scaled = (bias[0, :, 0, 0] * jnp.float32(1.0 / DEG)).reshape(1, C_OUT)

    h = _compute_h(x_pad, w_scaled, b_scaled)

    ei = edge_index[0, 0].astype(jnp.int32)
    ei_pad = jnp.pad(ei, ((0, N_PAD - N), (0, 0)))
    ei_groups = ei_pad.reshape(N_PAD // GROUP, GROUP * K)

    out_pad = _sc_aggregate(h, ei_groups)

    out = out_pad[:N].T
    return out.reshape(1, C_OUT, N, 1)
